# baseline (speedup 1.0000x reference)
import jax
import jax.numpy as jnp
from jax import lax
from jax.experimental import pallas as pl
from jax.experimental.pallas import tpu as pltpu
from jax.experimental.pallas import tpu_sc as plsc

B = 16384
EMB = 16
L = 16
NC = 2
NS = 16
NW = NC * NS
BPW = B // NW
NG = BPW // L


def _body(x_hbm, uid_hbm, iid_hbm, w_hbm, out_hbm,
          xv, uidx, iidx, urows, irows, outv, wv, sem_u, sem_i):
    wid = lax.axis_index("s") * NC + lax.axis_index("c")
    base = wid * BPW

    pltpu.sync_copy(x_hbm.at[pl.ds(base, BPW)], xv)
    pltpu.sync_copy(w_hbm, wv)

    umax = uid_hbm.shape[0] - 1
    imax = iid_hbm.shape[0] - 1
    lane = lax.iota(jnp.int32, L)
    zeros = jnp.zeros((L,), jnp.int32)
    ones = jnp.ones((L,), jnp.int32)

    for g in range(NG):
        rows = g * L + lane
        u = plsc.load_gather(xv, [rows, zeros])
        i = plsc.load_gather(xv, [rows, ones])
        u = jnp.minimum(jnp.maximum(u, 0), umax)
        i = jnp.minimum(jnp.maximum(i, 0), imax)
        uidx[pl.ds(g * L, L)] = u
        iidx[pl.ds(g * L, L)] = i

    cu = pltpu.async_copy(uid_hbm.at[uidx], urows, sem_u)
    ci = pltpu.async_copy(iid_hbm.at[iidx], irows, sem_i)
    cu.wait()
    ci.wait()

    wvec = wv[0]
    ws = [wvec[e] for e in range(EMB)]

    def compute(g, carry):
        rows = g * L + lane
        acc = jnp.zeros((L,), jnp.float32)
        for e in range(EMB):
            col = jnp.full((L,), e, jnp.int32)
            uv = plsc.load_gather(urows, [rows, col])
            iv = plsc.load_gather(irows, [rows, col])
            acc = acc + uv * iv * ws[e]
        outv[pl.ds(g * L, L)] = acc
        return carry

    lax.fori_loop(0, NG, compute, 0)

    pltpu.sync_copy(outv, out_hbm.at[pl.ds(base, BPW)])


def kernel(x, uid_table, iid_table, W):
    mesh = plsc.VectorSubcoreMesh(
        core_axis_name="c", subcore_axis_name="s",
        num_cores=NC, num_subcores=NS)
    f = pl.kernel(
        _body,
        out_type=jax.ShapeDtypeStruct((B,), jnp.float32),
        mesh=mesh,
        scratch_types=[
            pltpu.VMEM((BPW, 2), jnp.int32),
            pltpu.VMEM((BPW,), jnp.int32),
            pltpu.VMEM((BPW,), jnp.int32),
            pltpu.VMEM((BPW, EMB), jnp.float32),
            pltpu.VMEM((BPW, EMB), jnp.float32),
            pltpu.VMEM((BPW,), jnp.float32),
            pltpu.VMEM((1, EMB), jnp.float32),
            pltpu.SemaphoreType.DMA,
            pltpu.SemaphoreType.DMA,
        ],
        name="gmf_sc",
        compiler_params=pltpu.CompilerParams(
            needs_layout_passes=False, use_tc_tiling_on_sc=False),
    )
    return f(x, uid_table, iid_table, W)


# speedup vs baseline: 1.0094x; 1.0094x over previous
import jax
import jax.numpy as jnp
from jax import lax
from jax.experimental import pallas as pl
from jax.experimental.pallas import tpu as pltpu
from jax.experimental.pallas import tpu_sc as plsc

B = 16384
EMB = 16
L = 16
NC = 2
NS = 16
NW = NC * NS
BPW = B // NW
NG = BPW // L


def _body(uidx_hbm, iidx_hbm, uid_hbm, iid_hbm, w_hbm, out_hbm,
          uidxv, iidxv, urows, irows, outv, wv, sem_u, sem_i):
    wid = lax.axis_index("s") * NC + lax.axis_index("c")
    base = wid * BPW

    pltpu.sync_copy(uidx_hbm.at[pl.ds(base, BPW)], uidxv)
    pltpu.sync_copy(iidx_hbm.at[pl.ds(base, BPW)], iidxv)
    pltpu.sync_copy(w_hbm, wv)

    umax = uid_hbm.shape[0] - 1
    imax = iid_hbm.shape[0] - 1
    lane = lax.iota(jnp.int32, L)

    for g in range(NG):
        sl = pl.ds(g * L, L)
        uidxv[sl] = jnp.minimum(jnp.maximum(uidxv[sl], 0), umax)
        iidxv[sl] = jnp.minimum(jnp.maximum(iidxv[sl], 0), imax)

    cu = pltpu.async_copy(uid_hbm.at[uidxv], urows, sem_u)
    ci = pltpu.async_copy(iid_hbm.at[iidxv], irows, sem_i)
    cu.wait()
    ci.wait()

    wvec = wv[0]
    ws = [wvec[e] for e in range(EMB)]

    def compute(g, carry):
        rows = g * L + lane
        acc = jnp.zeros((L,), jnp.float32)
        for e in range(EMB):
            col = jnp.full((L,), e, jnp.int32)
            u = plsc.load_gather(urows, [rows, col])
            i = plsc.load_gather(irows, [rows, col])
            acc = acc + u * i * ws[e]
        outv[pl.ds(g * L, L)] = acc
        return carry

    lax.fori_loop(0, NG, compute, 0)

    pltpu.sync_copy(outv, out_hbm.at[pl.ds(base, BPW)])


def kernel(x, uid_table, iid_table, W):
    uidx = x[:, 0]
    iidx = x[:, 1]
    mesh = plsc.VectorSubcoreMesh(
        core_axis_name="c", subcore_axis_name="s",
        num_cores=NC, num_subcores=NS)
    f = pl.kernel(
        _body,
        out_type=jax.ShapeDtypeStruct((B,), jnp.float32),
        mesh=mesh,
        scratch_types=[
            pltpu.VMEM((BPW,), jnp.int32),
            pltpu.VMEM((BPW,), jnp.int32),
            pltpu.VMEM((BPW, EMB), jnp.float32),
            pltpu.VMEM((BPW, EMB), jnp.float32),
            pltpu.VMEM((BPW,), jnp.float32),
            pltpu.VMEM((1, EMB), jnp.float32),
            pltpu.SemaphoreType.DMA,
            pltpu.SemaphoreType.DMA,
        ],
        name="gmf_sc",
        compiler_params=pltpu.CompilerParams(
            needs_layout_passes=False, use_tc_tiling_on_sc=False),
    )
    return f(uidx, iidx, uid_table, iid_table, W)
